# Initial kernel scaffold; baseline (speedup 1.0000x reference)
import jax
import jax.numpy as jnp
from jax.experimental import pallas as pl


def kernel(atom_type, edge_index, edge_type, pos, batch, noise_level, d_noise, num_graphs, sigmas, params):
    raise NotImplementedError("write your pallas kernel here")


# speedup vs baseline: 2.8422x; 2.8422x over previous
import functools

import jax
import jax.numpy as jnp
from jax import lax
from jax.experimental import pallas as pl
from jax.experimental.pallas import tpu as pltpu
from jax.experimental.pallas import tpu_sc as plsc

N = 10000
E = 320000
H = 128
G = 128
NC = 2
NS = 16
NW = NC * NS
EPW = E // NW
CH = 80
NCHUNK = EPW // CH
NCB = 25
NSB = NCHUNK // NCB
NHC = 5008
ACC_R = NHC + 8
EPT = E // NS
NCH_C = EPT // CH
NSB_C = NCH_C // NCB
ZR = 312
ZCH_C = 104


def _sc_mesh():
    return plsc.VectorSubcoreMesh(
        core_axis_name="c", subcore_axis_name="s", num_cores=NC, num_subcores=NS
    )


_SC_PARAMS = pltpu.CompilerParams(needs_layout_passes=False)


def _edge_pre(row2, col2, batch, posx, posy, posz, sig64, noise_level):
    @functools.partial(
        pl.kernel,
        out_type=(
            jax.ShapeDtypeStruct((E,), jnp.float32),
            jax.ShapeDtypeStruct((E,), jnp.float32),
            jax.ShapeDtypeStruct((E,), jnp.int32),
        ),
        mesh=_sc_mesh(),
        compiler_params=_SC_PARAMS,
        scratch_types=[
            pltpu.VMEM((NCB, CH), jnp.int32),
            pltpu.VMEM((NCB, CH), jnp.int32),
            pltpu.VMEM((N,), jnp.int32),
            pltpu.VMEM((N,), jnp.float32),
            pltpu.VMEM((N,), jnp.float32),
            pltpu.VMEM((N,), jnp.float32),
            pltpu.VMEM((64,), jnp.float32),
            pltpu.VMEM((G,), jnp.int32),
            pltpu.VMEM((G,), jnp.float32),
            pltpu.VMEM((NCB * CH,), jnp.float32),
            pltpu.VMEM((NCB * CH,), jnp.float32),
            pltpu.VMEM((NCB * CH,), jnp.int32),
        ],
    )
    def k(row_h, col_h, batch_h, px_h, py_h, pz_h, sig_h, nl_h,
          d2_h, se_h, eg_h,
          rows_v, cols_v, batch_v, px_v, py_v, pz_v, sig_v, nl_v, sg_v,
          d2_v, se_v, eg_v):
        wid = lax.axis_index("s") * NC + lax.axis_index("c")
        ebase = wid * EPW
        pltpu.sync_copy(batch_h, batch_v)
        pltpu.sync_copy(px_h, px_v)
        pltpu.sync_copy(py_h, py_v)
        pltpu.sync_copy(pz_h, pz_v)
        pltpu.sync_copy(sig_h, sig_v)
        pltpu.sync_copy(nl_h, nl_v)
        for i in range(G // 16):
            nl = nl_v[pl.ds(16 * i, 16)]
            sg_v[pl.ds(16 * i, 16)] = plsc.load_gather(sig_v, [nl])

        def outer(sb, carry):
            pltpu.sync_copy(row_h.at[wid, sb], rows_v)
            pltpu.sync_copy(col_h.at[wid, sb], cols_v)

            def body(j, c2):
                for kk in range(CH // 16):
                    o = j * CH + 16 * kk
                    r = rows_v[j, pl.ds(16 * kk, 16)]
                    c = cols_v[j, pl.ds(16 * kk, 16)]
                    eg = plsc.load_gather(batch_v, [r])
                    se = plsc.load_gather(sg_v, [eg])
                    dx = plsc.load_gather(px_v, [r]) - plsc.load_gather(px_v, [c])
                    dy = plsc.load_gather(py_v, [r]) - plsc.load_gather(py_v, [c])
                    dz = plsc.load_gather(pz_v, [r]) - plsc.load_gather(pz_v, [c])
                    d2_v[pl.ds(o, 16)] = dx * dx + dy * dy + dz * dz
                    se_v[pl.ds(o, 16)] = se
                    eg_v[pl.ds(o, 16)] = eg
                return c2

            lax.fori_loop(0, NCB, body, 0)
            ob = ebase + sb * NCB * CH
            pltpu.sync_copy(d2_v, d2_h.at[pl.ds(ob, NCB * CH)])
            pltpu.sync_copy(se_v, se_h.at[pl.ds(ob, NCB * CH)])
            pltpu.sync_copy(eg_v, eg_h.at[pl.ds(ob, NCB * CH)])
            return carry

        lax.fori_loop(0, NSB, outer, 0)

    return k(row2, col2, batch, posx, posy, posz, sig64, noise_level)


def _conv_sc(hidden, eattr, rowc, colc):
    @functools.partial(
        pl.kernel,
        out_type=jax.ShapeDtypeStruct((N, H), jnp.float32),
        mesh=_sc_mesh(),
        compiler_params=_SC_PARAMS,
        scratch_types=[
            pltpu.VMEM((NCB, CH), jnp.int32),
            pltpu.VMEM((NCB, CH), jnp.int32),
            pltpu.VMEM((CH,), jnp.int32),
            pltpu.VMEM((CH, H), jnp.float32),
            pltpu.VMEM((CH, H), jnp.float32),
            pltpu.VMEM((ZCH_C, H), jnp.float32),
            pltpu.VMEM_SHARED((ACC_R, H), jnp.float32),
        ],
    )
    def k(hid_h, ea_h, row_h, col_h, out_h,
          rows_v, cols_v, idx_v, g_v, e_v, z_v, acc):
        cid = lax.axis_index("c")
        sid = lax.axis_index("s")
        nbase = cid * NHC

        def zb(r, carry):
            for c8 in range(H // 16):
                z_v[r, pl.ds(16 * c8, 16)] = jnp.zeros((16,), jnp.float32)
            return carry

        lax.fori_loop(0, ZCH_C, zb, 0)
        t0 = sid * ZR
        for zz in range(ZR // ZCH_C):
            pltpu.sync_copy(z_v, acc.at[pl.ds(t0 + zz * ZCH_C, ZCH_C)])

        @pl.when(sid == NS - 1)
        def _():
            pltpu.sync_copy(z_v.at[pl.ds(0, ACC_R - NS * ZR)],
                            acc.at[pl.ds(NS * ZR, ACC_R - NS * ZR)])

        plsc.subcore_barrier()

        def outer(sb, carry):
            pltpu.sync_copy(row_h.at[sid, sb], rows_v)
            pltpu.sync_copy(col_h.at[sid, sb], cols_v)

            def body(j, c3):
                pltpu.sync_copy(hid_h.at[rows_v.at[j]], g_v)
                pltpu.sync_copy(
                    ea_h.at[pl.ds(sid * EPT + (sb * NCB + j) * CH, CH)], e_v)
                for kk in range(CH // 16):
                    c = cols_v[j, pl.ds(16 * kk, 16)]
                    c2 = c - nbase
                    inb = jnp.logical_and(c2 >= 0, c2 < NHC)
                    idx_v[pl.ds(16 * kk, 16)] = jnp.where(inb, c2, NHC)

                def rb(r, c4):
                    for c8 in range(H // 16):
                        s = pl.ds(16 * c8, 16)
                        g_v[r, s] = jnp.maximum(g_v[r, s] + e_v[r, s], 0.0)
                    return c4

                lax.fori_loop(0, CH, rb, 0)
                pltpu.sync_copy(g_v, acc.at[idx_v], add=True)
                return c3

            lax.fori_loop(0, NCB, body, 0)
            return carry

        lax.fori_loop(0, NSB_C, outer, 0)
        plsc.subcore_barrier()
        pltpu.sync_copy(acc.at[pl.ds(t0, ZR)], out_h.at[pl.ds(nbase + t0, ZR)])

        @pl.when(jnp.logical_and(sid == NS - 1, cid == 0))
        def _():
            pltpu.sync_copy(acc.at[pl.ds(NS * ZR, NHC - NS * ZR)],
                            out_h.at[pl.ds(NS * ZR, NHC - NS * ZR)])

    return k(hidden, eattr, rowc, colc)


def _pair_sc(hidden, row2, col2):
    @functools.partial(
        pl.kernel,
        out_type=jax.ShapeDtypeStruct((E, H), jnp.float32),
        mesh=_sc_mesh(),
        compiler_params=_SC_PARAMS,
        scratch_types=[
            pltpu.VMEM((NCB, CH), jnp.int32),
            pltpu.VMEM((NCB, CH), jnp.int32),
            pltpu.VMEM((CH, H), jnp.float32),
            pltpu.VMEM((CH, H), jnp.float32),
        ],
    )
    def k(hid_h, row_h, col_h, out_h, rows_v, cols_v, a_v, b_v):
        wid = lax.axis_index("s") * NC + lax.axis_index("c")

        def outer(sb, carry):
            pltpu.sync_copy(row_h.at[wid, sb], rows_v)
            pltpu.sync_copy(col_h.at[wid, sb], cols_v)

            def body(j, c3):
                pltpu.sync_copy(hid_h.at[rows_v.at[j]], a_v)
                pltpu.sync_copy(hid_h.at[cols_v.at[j]], b_v)

                def rb(r, c2):
                    for c8 in range(H // 16):
                        s = pl.ds(16 * c8, 16)
                        a_v[r, s] = a_v[r, s] * b_v[r, s]
                    return c2

                lax.fori_loop(0, CH, rb, 0)
                pltpu.sync_copy(
                    a_v, out_h.at[pl.ds(wid * EPW + (sb * NCB + j) * CH, CH)])
                return c3

            lax.fori_loop(0, NCB, body, 0)
            return carry

        lax.fori_loop(0, NSB, outer, 0)

    return k(hidden, row2, col2)


def _node_init(atom_type2, emb_pad):
    Nb = 2000

    def body(at_r, em_r, o_r):
        io = lax.broadcasted_iota(jnp.int32, (Nb, 128), 1)
        oh = (at_r[...] == io).astype(jnp.float32)
        o_r[...] = jnp.dot(oh, em_r[...], preferred_element_type=jnp.float32)

    return pl.pallas_call(
        body,
        grid=(N // Nb,),
        in_specs=[
            pl.BlockSpec((Nb, 1), lambda i: (i, 0)),
            pl.BlockSpec((128, H), lambda i: (0, 0)),
        ],
        out_specs=pl.BlockSpec((Nb, H), lambda i: (i, 0)),
        out_shape=jax.ShapeDtypeStruct((N, H), jnp.float32),
    )(atom_type2, emb_pad)


def _edge_mlp(d2, sig, dn, et, w1, b1, w2, b2, emb_pad):
    Eb = 512

    def body(d2_r, sg_r, dn_r, et_r, w1_r, b1_r, w2_r, b2_r, em_r, o_r):
        d = jnp.sqrt(d2_r[...])
        pert = d + dn_r[...] * sg_r[...]
        t1 = jnp.maximum(pert * w1_r[...] + b1_r[...], 0.0)
        demb = jnp.dot(t1, w2_r[...], preferred_element_type=jnp.float32) + b2_r[...]
        io = lax.broadcasted_iota(jnp.int32, (Eb, 128), 1)
        oh = (et_r[...] == io).astype(jnp.float32)
        ea = jnp.dot(oh, em_r[...], preferred_element_type=jnp.float32)
        o_r[...] = demb * ea

    eb = lambda i: (i, 0)
    cb = lambda i: (0, 0)
    return pl.pallas_call(
        body,
        grid=(E // Eb,),
        in_specs=[
            pl.BlockSpec((Eb, 1), eb),
            pl.BlockSpec((Eb, 1), eb),
            pl.BlockSpec((Eb, 1), eb),
            pl.BlockSpec((Eb, 1), eb),
            pl.BlockSpec((1, H), cb),
            pl.BlockSpec((1, H), cb),
            pl.BlockSpec((H, H), cb),
            pl.BlockSpec((1, H), cb),
            pl.BlockSpec((128, H), cb),
        ],
        out_specs=pl.BlockSpec((Eb, H), eb),
        out_shape=jax.ShapeDtypeStruct((E, H), jnp.float32),
    )(d2, sig, dn, et, w1, b1, w2, b2, emb_pad)


def _conv_mlp(hidden, agg, epsv, w1, b1, w2, b2):
    Nb = 2000

    def body(h_r, a_r, ep_r, w1_r, b1_r, w2_r, b2_r, o_r):
        x = ep_r[...] * h_r[...] + a_r[...]
        t = jnp.maximum(
            jnp.dot(x, w1_r[...], preferred_element_type=jnp.float32) + b1_r[...], 0.0
        )
        o_r[...] = jnp.dot(t, w2_r[...], preferred_element_type=jnp.float32) + b2_r[...]

    nb = lambda i: (i, 0)
    cb = lambda i: (0, 0)
    return pl.pallas_call(
        body,
        grid=(N // Nb,),
        in_specs=[
            pl.BlockSpec((Nb, H), nb),
            pl.BlockSpec((Nb, H), nb),
            pl.BlockSpec((1, H), cb),
            pl.BlockSpec((H, H), cb),
            pl.BlockSpec((1, H), cb),
            pl.BlockSpec((H, H), cb),
            pl.BlockSpec((1, H), cb),
        ],
        out_specs=pl.BlockSpec((Nb, H), nb),
        out_shape=jax.ShapeDtypeStruct((N, H), jnp.float32),
    )(hidden, agg, epsv, w1, b1, w2, b2)


def _out_loss(hprod, ea, dn, eg, wa, wb, b1, w2, b2, w3, b3):
    Eb = 512

    def body(hp_r, ea_r, dn_r, eg_r, wa_r, wb_r, b1_r, w2_r, b2_r, w3_r, b3_r, o_r):
        i = pl.program_id(0)
        x = jnp.maximum(
            jnp.dot(hp_r[...], wa_r[...], preferred_element_type=jnp.float32)
            + jnp.dot(ea_r[...], wb_r[...], preferred_element_type=jnp.float32)
            + b1_r[...],
            0.0,
        )
        y = jnp.maximum(
            jnp.dot(x, w2_r[...], preferred_element_type=jnp.float32) + b2_r[...], 0.0
        )
        s = jnp.sum(y * w3_r[...], axis=1, keepdims=True) + b3_r[...]
        le = 0.5 * (s + dn_r[...]) ** 2
        io = lax.broadcasted_iota(jnp.int32, (Eb, G), 1)
        contrib = jnp.sum(jnp.where(eg_r[...] == io, le, 0.0), axis=0, keepdims=True)

        @pl.when(i == 0)
        def _():
            o_r[...] = jnp.zeros_like(o_r)

        o_r[...] += contrib

    eb = lambda i: (i, 0)
    cb = lambda i: (0, 0)
    return pl.pallas_call(
        body,
        grid=(E // Eb,),
        in_specs=[
            pl.BlockSpec((Eb, H), eb),
            pl.BlockSpec((Eb, H), eb),
            pl.BlockSpec((Eb, 1), eb),
            pl.BlockSpec((Eb, 1), eb),
            pl.BlockSpec((H, H), cb),
            pl.BlockSpec((H, H), cb),
            pl.BlockSpec((1, H), cb),
            pl.BlockSpec((H, H // 2), cb),
            pl.BlockSpec((1, H // 2), cb),
            pl.BlockSpec((1, H // 2), cb),
            pl.BlockSpec((1, 1), cb),
        ],
        out_specs=pl.BlockSpec((1, G), cb),
        out_shape=jax.ShapeDtypeStruct((1, G), jnp.float32),
    )(hprod, ea, dn, eg, wa, wb, b1, w2, b2, w3, b3)


def kernel(atom_type, edge_index, edge_type, pos, batch, noise_level, d_noise,
           num_graphs, sigmas, params):
    f32 = jnp.float32
    i32 = jnp.int32
    row = edge_index[0].astype(i32)
    col = edge_index[1].astype(i32)
    row2 = row.reshape(NW, NSB, NCB, CH)
    col2 = col.reshape(NW, NSB, NCB, CH)
    rowc = row.reshape(NS, NSB_C, NCB, CH)
    colc = col.reshape(NS, NSB_C, NCB, CH)
    posx = pos[:, 0].astype(f32)
    posy = pos[:, 1].astype(f32)
    posz = pos[:, 2].astype(f32)
    sig64 = jnp.zeros((64,), f32).at[: sigmas.shape[0]].set(sigmas.astype(f32))

    d2, sig_e, e2g = _edge_pre(
        row2, col2, batch.astype(i32), posx, posy, posz, sig64,
        noise_level.astype(i32)
    )

    emb_n = jnp.zeros((128, H), f32).at[:100].set(params["node_emb"])
    emb_e = jnp.zeros((128, H), f32).at[:100].set(params["edge_emb"])
    hidden = _node_init(atom_type.astype(i32).reshape(N, 1), emb_n)

    im = params["input_mlp"]
    eattr = _edge_mlp(
        d2.reshape(E, 1), sig_e.reshape(E, 1), d_noise.astype(f32),
        edge_type.astype(i32).reshape(E, 1),
        im[0]["W"], im[0]["b"][None], im[1]["W"], im[1]["b"][None], emb_e,
    )

    for conv in params["convs"]:
        agg = _conv_sc(hidden, eattr, rowc, colc)
        epsv = jnp.broadcast_to(1.0 + conv["eps"], (1, H)).astype(f32)
        m = conv["mlp"]
        hidden = _conv_mlp(
            hidden, agg, epsv,
            m[0]["W"], m[0]["b"][None], m[1]["W"], m[1]["b"][None],
        )

    hprod = _pair_sc(hidden, row2, col2)

    om = params["output_mlp"]
    wa = om[0]["W"][:H]
    wb = om[0]["W"][H:]
    loss = _out_loss(
        hprod, eattr, d_noise.astype(f32), e2g.reshape(E, 1),
        wa, wb, om[0]["b"][None], om[1]["W"], om[1]["b"][None],
        om[2]["W"].reshape(1, H // 2), om[2]["b"].reshape(1, 1),
    )
    return loss.reshape(G)
